# baseline (speedup 1.0000x reference)
import functools

import jax
import jax.numpy as jnp
from jax import lax
from jax.experimental import pallas as pl
from jax.experimental.pallas import tpu as pltpu
from jax.experimental.pallas import tpu_sc as plsc

NSEG = 512
N = 100000
D = 512
NC = 2
NS = 16
NW = NC * NS
L = 16
PER_W = 3200
PAD_N = NW * PER_W
HBINS = 640
OFFS = 544
CHUNK = 80
CUR = 17

_mesh = plsc.VectorSubcoreMesh(
    core_axis_name="c", subcore_axis_name="s", num_cores=NC, num_subcores=NS
)
_params = pltpu.CompilerParams(needs_layout_passes=False)


def _wid():
    return lax.axis_index("s") * NC + lax.axis_index("c")


TAIL = N - (NW - 1) * PER_W


@functools.partial(
    pl.kernel,
    out_type=jax.ShapeDtypeStruct((NW * HBINS,), jnp.int32),
    mesh=_mesh,
    scratch_types=[
        pltpu.VMEM((PER_W,), jnp.int32),
        pltpu.VMEM((HBINS,), jnp.int32),
    ],
    compiler_params=_params,
)
def _hist_kernel(batch_hbm, out_hbm, bvec, hist):
    w = _wid()

    @pl.when(w < NW - 1)
    def _full():
        pltpu.sync_copy(
            batch_hbm.at[pl.ds(pl.multiple_of(w * PER_W, 128), PER_W)], bvec
        )

    @pl.when(w == NW - 1)
    def _tail():
        pltpu.sync_copy(
            batch_hbm.at[pl.ds((NW - 1) * PER_W, TAIL)], bvec.at[pl.ds(0, TAIL)]
        )

    zeros = jnp.zeros((L,), jnp.int32)
    for j in range(HBINS // L):
        hist[pl.ds(j * L, L)] = zeros
    ones = jnp.ones((L,), jnp.int32)

    def body(i, carry):
        idx = bvec[pl.ds(i * L, L)]
        plsc.addupdate_scatter(hist, [idx], ones)
        return carry

    nvec = jnp.where(w == NW - 1, TAIL // L, PER_W // L)
    lax.fori_loop(0, nvec, body, 0)
    pltpu.sync_copy(hist, out_hbm.at[pl.ds(pl.multiple_of(w * HBINS, 128), HBINS)])


@functools.partial(
    pl.kernel,
    out_type=jax.ShapeDtypeStruct((NSEG, 2 * D), jnp.float32),
    mesh=_mesh,
    scratch_types=[
        pltpu.VMEM((NW * HBINS,), jnp.int32),
        pltpu.VMEM((OFFS,), jnp.int32),
        pltpu.VMEM((L, 2 * D), jnp.float32),
        pltpu.VMEM((CHUNK, D), jnp.float32),
        pltpu.VMEM((CHUNK, D), jnp.float32),
        pltpu.SMEM((32,), jnp.int32),
        pltpu.SemaphoreType.DMA,
        pltpu.SemaphoreType.DMA,
    ],
    compiler_params=_params,
)
def _main_kernel(x_hbm, hist_hbm, out_hbm, table, offs, acc, buf0, buf1, sm, sem0, sem1):
    w = _wid()
    pltpu.sync_copy(hist_hbm, table)

    zeros_i = jnp.zeros((L,), jnp.int32)
    carry = jnp.int32(0)
    for blk in range(OFFS // L - 1):
        def add_row(ww, tot):
            return tot + table[pl.ds(ww * HBINS + blk * L, L)]

        tot = lax.fori_loop(0, NW, add_row, zeros_i)
        cs = plsc.cumsum(tot)
        offs[pl.ds(blk * L, L)] = cs - tot + carry
        carry = carry + jnp.sum(tot)

    seg0 = w * L
    va = offs[pl.ds(seg0, L)]
    vb = offs[pl.ds(seg0 + L, L)]
    iota = lax.iota(jnp.int32, L)

    def pick(vec, j):
        return jnp.sum(jnp.where(iota == j, vec, 0))

    for j in range(L):
        sm[j] = pick(va, j)
    sm[L] = pick(vb, 0)
    sm[CUR] = 0

    wlo = sm[0]
    whi = sm[L]
    first = (wlo // 8) * 8
    nchunks = (whi - first + CHUNK - 1) // CHUNK

    def issue(k, buf, sem):
        gbase = first + k * CHUNK
        base = jnp.minimum(gbase, N - CHUNK)
        pltpu.async_copy(x_hbm.at[pl.ds(pl.multiple_of(base, 8), CHUNK)], buf, sem)

    @pl.when(nchunks > 0)
    def _prime():
        issue(0, buf0, sem0)

    zeros_f = jnp.zeros((L,), jnp.float32)
    ninf_f = jnp.full((L,), -jnp.inf, jnp.float32)

    def init_body(s, c):
        for j in range(D // L):
            acc[s, pl.ds(j * L, L)] = zeros_f
        for j in range(D // L):
            acc[s, pl.ds(D + j * L, L)] = ninf_f
        return c

    lax.fori_loop(0, L, init_body, 0)

    def process(k, buf):
        gbase = first + k * CHUNK
        base = jnp.minimum(gbase, N - CHUNK)
        cend = jnp.minimum(gbase + CHUNK, whi)

        def cond(cv):
            cur, done = cv
            return jnp.logical_and(
                jnp.logical_not(done),
                jnp.logical_and(cur < L, sm[cur] < cend),
            )

        def body(cv):
            cur, done = cv
            lo_s = jnp.maximum(sm[cur], gbase)
            hi_s = jnp.minimum(sm[cur + 1], cend)
            jlo = lo_s - base
            jhi = hi_s - base
            npairs = (jhi - jlo) // 2
            for h in range(2):
                half = h * (D // 2)

                def row_body(j, rcv):
                    sums, maxs = rcv
                    ns, nm = [], []
                    for cc in range(D // 2 // L):
                        v = buf[j, pl.ds(half + cc * L, L)]
                        ns.append(sums[cc] + v)
                        nm.append(jnp.maximum(maxs[cc], v))
                    return tuple(ns), tuple(nm)

                def pair_rows(i, rcv):
                    j = jlo + 2 * i
                    return row_body(j + 1, row_body(j, rcv))

                init = (
                    tuple(zeros_f for _ in range(D // 2 // L)),
                    tuple(ninf_f for _ in range(D // 2 // L)),
                )
                sums, maxs = lax.fori_loop(0, npairs, pair_rows, init)
                sums, maxs = lax.fori_loop(
                    jlo + 2 * npairs, jhi, row_body, (sums, maxs)
                )
                for cc in range(D // 2 // L):
                    dss = pl.ds(half + cc * L, L)
                    acc[cur, dss] = acc[cur, dss] + sums[cc]
                    dsm = pl.ds(D + half + cc * L, L)
                    acc[cur, dsm] = jnp.maximum(acc[cur, dsm], maxs[cc])
            adv = sm[cur + 1] <= cend
            return cur + adv.astype(jnp.int32), jnp.logical_not(adv)

        cur0 = sm[CUR]
        curf, _ = lax.while_loop(cond, body, (cur0, jnp.bool_(False)))
        sm[CUR] = curf

    def wait(buf, sem):
        pltpu.make_async_copy(x_hbm.at[pl.ds(0, CHUNK)], buf, sem).wait()

    npairs = (nchunks + 1) // 2

    def pair_body(g, c):
        k0 = 2 * g

        @pl.when(k0 < nchunks)
        def _b0():
            wait(buf0, sem0)

            @pl.when(k0 + 1 < nchunks)
            def _i1():
                issue(k0 + 1, buf1, sem1)

            process(k0, buf0)

        @pl.when(k0 + 1 < nchunks)
        def _b1():
            wait(buf1, sem1)

            @pl.when(k0 + 2 < nchunks)
            def _i2():
                issue(k0 + 2, buf0, sem0)

            process(k0 + 1, buf1)

        return c

    lax.fori_loop(0, npairs, pair_body, 0)

    def fin_body(s, c):
        n = sm[s + 1] - sm[s]
        nf = jnp.broadcast_to(jnp.maximum(n, 1).astype(jnp.float32), (L,))
        scale = jnp.where(n > 0, 1.0, 0.0) / nf
        for j in range(D // L):
            dss = pl.ds(j * L, L)
            acc[s, dss] = acc[s, dss] * scale
        for j in range(D // L):
            dsm = pl.ds(D + j * L, L)
            acc[s, dsm] = jnp.where(n > 0, acc[s, dsm], 0.0)
        return c

    lax.fori_loop(0, L, fin_body, 0)
    pltpu.sync_copy(acc, out_hbm.at[pl.ds(pl.multiple_of(seg0, 8), L)])


def kernel(x, batch):
    batch = batch.astype(jnp.int32)
    hist = _hist_kernel(batch)
    return _main_kernel(x, hist)


# speedup vs baseline: 1.0129x; 1.0129x over previous
import functools

import jax
import jax.numpy as jnp
from jax import lax
from jax.experimental import pallas as pl
from jax.experimental.pallas import tpu as pltpu
from jax.experimental.pallas import tpu_sc as plsc

NSEG = 512
N = 100000
D = 512
NC = 2
NS = 16
NW = NC * NS
L = 16
PER_W = 3200
PAD_N = NW * PER_W
HBINS = 640
OFFS = 544
CHUNK = 80
CUR = 17

_mesh = plsc.VectorSubcoreMesh(
    core_axis_name="c", subcore_axis_name="s", num_cores=NC, num_subcores=NS
)
_params = pltpu.CompilerParams(needs_layout_passes=False)


def _wid():
    return lax.axis_index("s") * NC + lax.axis_index("c")


TAIL = N - (NW - 1) * PER_W


@functools.partial(
    pl.kernel,
    out_type=jax.ShapeDtypeStruct((NW * HBINS,), jnp.int32),
    mesh=_mesh,
    scratch_types=[
        pltpu.VMEM((PER_W,), jnp.int32),
        pltpu.VMEM((HBINS,), jnp.int32),
    ],
    compiler_params=_params,
)
def _hist_kernel(batch_hbm, out_hbm, bvec, hist):
    w = _wid()

    @pl.when(w < NW - 1)
    def _full():
        pltpu.sync_copy(
            batch_hbm.at[pl.ds(pl.multiple_of(w * PER_W, 128), PER_W)], bvec
        )

    @pl.when(w == NW - 1)
    def _tail():
        pltpu.sync_copy(
            batch_hbm.at[pl.ds((NW - 1) * PER_W, TAIL)], bvec.at[pl.ds(0, TAIL)]
        )

    zeros = jnp.zeros((L,), jnp.int32)
    for j in range(HBINS // L):
        hist[pl.ds(j * L, L)] = zeros
    ones = jnp.ones((L,), jnp.int32)

    def body(i, carry):
        idx = bvec[pl.ds(i * L, L)]
        plsc.addupdate_scatter(hist, [idx], ones)
        return carry

    nvec = jnp.where(w == NW - 1, TAIL // L, PER_W // L)
    lax.fori_loop(0, nvec, body, 0)
    pltpu.sync_copy(hist, out_hbm.at[pl.ds(pl.multiple_of(w * HBINS, 128), HBINS)])


@functools.partial(
    pl.kernel,
    out_type=jax.ShapeDtypeStruct((NSEG, 2 * D), jnp.float32),
    mesh=_mesh,
    scratch_types=[
        pltpu.VMEM((NW * HBINS,), jnp.int32),
        pltpu.VMEM((OFFS,), jnp.int32),
        pltpu.VMEM((L, 2 * D), jnp.float32),
        pltpu.VMEM((CHUNK, D), jnp.float32),
        pltpu.VMEM((CHUNK, D), jnp.float32),
        pltpu.SMEM((32,), jnp.int32),
        pltpu.SemaphoreType.DMA,
        pltpu.SemaphoreType.DMA,
    ],
    compiler_params=_params,
)
def _main_kernel(x_hbm, hist_hbm, out_hbm, table, offs, acc, buf0, buf1, sm, sem0, sem1):
    w = _wid()
    pltpu.sync_copy(hist_hbm, table)

    zeros_i = jnp.zeros((L,), jnp.int32)
    carry = jnp.int32(0)
    for blk in range(OFFS // L - 1):
        def add_row(ww, tot):
            return tot + table[pl.ds(ww * HBINS + blk * L, L)]

        tot = lax.fori_loop(0, NW, add_row, zeros_i)
        cs = plsc.cumsum(tot)
        offs[pl.ds(blk * L, L)] = cs - tot + carry
        carry = carry + jnp.sum(tot)

    seg0 = w * L
    va = offs[pl.ds(seg0, L)]
    vb = offs[pl.ds(seg0 + L, L)]
    iota = lax.iota(jnp.int32, L)

    def pick(vec, j):
        return jnp.sum(jnp.where(iota == j, vec, 0))

    for j in range(L):
        sm[j] = pick(va, j)
    sm[L] = pick(vb, 0)
    sm[CUR] = 0

    wlo = sm[0]
    whi = sm[L]
    first = (wlo // 8) * 8
    nchunks = (whi - first + CHUNK - 1) // CHUNK

    def issue(k, buf, sem):
        gbase = first + k * CHUNK
        base = jnp.minimum(gbase, N - CHUNK)
        pltpu.async_copy(x_hbm.at[pl.ds(pl.multiple_of(base, 8), CHUNK)], buf, sem)

    @pl.when(nchunks > 0)
    def _prime():
        issue(0, buf0, sem0)

    zeros_f = jnp.zeros((L,), jnp.float32)
    ninf_f = jnp.full((L,), -jnp.inf, jnp.float32)

    def init_body(s, c):
        for j in range(D // L):
            acc[s, pl.ds(j * L, L)] = zeros_f
        for j in range(D // L):
            acc[s, pl.ds(D + j * L, L)] = ninf_f
        return c

    lax.fori_loop(0, L, init_body, 0)

    def process(k, buf):
        gbase = first + k * CHUNK
        base = jnp.minimum(gbase, N - CHUNK)
        cend = jnp.minimum(gbase + CHUNK, whi)

        def cond(cv):
            cur, done = cv
            return jnp.logical_and(
                jnp.logical_not(done),
                jnp.logical_and(cur < L, sm[cur] < cend),
            )

        def body(cv):
            cur, done = cv
            lo_s = jnp.maximum(sm[cur], gbase)
            hi_s = jnp.minimum(sm[cur + 1], cend)
            jlo = lo_s - base
            jhi = hi_s - base
            for h in range(2):
                half = h * (D // 2)

                def row_body(j, rcv):
                    sums, maxs = rcv
                    ns, nm = [], []
                    for cc in range(D // 2 // L):
                        v = buf[j, pl.ds(half + cc * L, L)]
                        ns.append(sums[cc] + v)
                        nm.append(jnp.maximum(maxs[cc], v))
                    return tuple(ns), tuple(nm)

                init = (
                    tuple(zeros_f for _ in range(D // 2 // L)),
                    tuple(ninf_f for _ in range(D // 2 // L)),
                )
                sums, maxs = lax.fori_loop(jlo, jhi, row_body, init)
                for cc in range(D // 2 // L):
                    dss = pl.ds(half + cc * L, L)
                    acc[cur, dss] = acc[cur, dss] + sums[cc]
                    dsm = pl.ds(D + half + cc * L, L)
                    acc[cur, dsm] = jnp.maximum(acc[cur, dsm], maxs[cc])
            adv = sm[cur + 1] <= cend
            return cur + adv.astype(jnp.int32), jnp.logical_not(adv)

        cur0 = sm[CUR]
        curf, _ = lax.while_loop(cond, body, (cur0, jnp.bool_(False)))
        sm[CUR] = curf

    def wait(buf, sem):
        pltpu.make_async_copy(x_hbm.at[pl.ds(0, CHUNK)], buf, sem).wait()

    npairs = (nchunks + 1) // 2

    def pair_body(g, c):
        k0 = 2 * g

        @pl.when(k0 < nchunks)
        def _b0():
            wait(buf0, sem0)

            @pl.when(k0 + 1 < nchunks)
            def _i1():
                issue(k0 + 1, buf1, sem1)

            process(k0, buf0)

        @pl.when(k0 + 1 < nchunks)
        def _b1():
            wait(buf1, sem1)

            @pl.when(k0 + 2 < nchunks)
            def _i2():
                issue(k0 + 2, buf0, sem0)

            process(k0 + 1, buf1)

        return c

    lax.fori_loop(0, npairs, pair_body, 0)

    def fin_body(s, c):
        n = sm[s + 1] - sm[s]
        nf = jnp.broadcast_to(jnp.maximum(n, 1).astype(jnp.float32), (L,))
        scale = jnp.where(n > 0, 1.0, 0.0) / nf
        for j in range(D // L):
            dss = pl.ds(j * L, L)
            acc[s, dss] = acc[s, dss] * scale
        for j in range(D // L):
            dsm = pl.ds(D + j * L, L)
            acc[s, dsm] = jnp.where(n > 0, acc[s, dsm], 0.0)
        return c

    lax.fori_loop(0, L, fin_body, 0)
    pltpu.sync_copy(acc, out_hbm.at[pl.ds(pl.multiple_of(seg0, 8), L)])


def kernel(x, batch):
    batch = batch.astype(jnp.int32)
    hist = _hist_kernel(batch)
    return _main_kernel(x, hist)


# speedup vs baseline: 1.0140x; 1.0010x over previous
import functools

import jax
import jax.numpy as jnp
from jax import lax
from jax.experimental import pallas as pl
from jax.experimental.pallas import tpu as pltpu
from jax.experimental.pallas import tpu_sc as plsc

NSEG = 512
N = 100000
D = 512
NC = 2
NS = 16
NW = NC * NS
L = 16
PER_W = 3200
PAD_N = NW * PER_W
HBINS = 640
OFFS = 544
CHUNK = 80
CUR = 17

_mesh = plsc.VectorSubcoreMesh(
    core_axis_name="c", subcore_axis_name="s", num_cores=NC, num_subcores=NS
)
_params = pltpu.CompilerParams(needs_layout_passes=False)


def _wid():
    return lax.axis_index("s") * NC + lax.axis_index("c")


TAIL = N - (NW - 1) * PER_W


@functools.partial(
    pl.kernel,
    out_type=jax.ShapeDtypeStruct((NW * HBINS,), jnp.int32),
    mesh=_mesh,
    scratch_types=[
        pltpu.VMEM((PER_W,), jnp.int32),
        pltpu.VMEM((HBINS,), jnp.int32),
    ],
    compiler_params=_params,
)
def _hist_kernel(batch_hbm, out_hbm, bvec, hist):
    w = _wid()

    @pl.when(w < NW - 1)
    def _full():
        pltpu.sync_copy(
            batch_hbm.at[pl.ds(pl.multiple_of(w * PER_W, 128), PER_W)], bvec
        )

    @pl.when(w == NW - 1)
    def _tail():
        pltpu.sync_copy(
            batch_hbm.at[pl.ds((NW - 1) * PER_W, TAIL)], bvec.at[pl.ds(0, TAIL)]
        )

    zeros = jnp.zeros((L,), jnp.int32)
    for j in range(HBINS // L):
        hist[pl.ds(j * L, L)] = zeros
    ones = jnp.ones((L,), jnp.int32)

    def body4(i, carry):
        for u in range(4):
            idx = bvec[pl.ds((i * 4 + u) * L, L)]
            plsc.addupdate_scatter(hist, [idx], ones)
        return carry

    def body(i, carry):
        idx = bvec[pl.ds(i * L, L)]
        plsc.addupdate_scatter(hist, [idx], ones)
        return carry

    nvec = jnp.where(w == NW - 1, TAIL // L, PER_W // L)
    lax.fori_loop(0, nvec // 4, body4, 0)
    lax.fori_loop((nvec // 4) * 4, nvec, body, 0)
    pltpu.sync_copy(hist, out_hbm.at[pl.ds(pl.multiple_of(w * HBINS, 128), HBINS)])


@functools.partial(
    pl.kernel,
    out_type=jax.ShapeDtypeStruct((NSEG, 2 * D), jnp.float32),
    mesh=_mesh,
    scratch_types=[
        pltpu.VMEM((NW * HBINS,), jnp.int32),
        pltpu.VMEM((OFFS,), jnp.int32),
        pltpu.VMEM((L, 2 * D), jnp.float32),
        pltpu.VMEM((CHUNK, D), jnp.float32),
        pltpu.VMEM((CHUNK, D), jnp.float32),
        pltpu.SMEM((32,), jnp.int32),
        pltpu.SemaphoreType.DMA,
        pltpu.SemaphoreType.DMA,
    ],
    compiler_params=_params,
)
def _main_kernel(x_hbm, hist_hbm, out_hbm, table, offs, acc, buf0, buf1, sm, sem0, sem1):
    w = _wid()
    pltpu.async_copy(hist_hbm, table, sem0)

    zeros_f = jnp.zeros((L,), jnp.float32)
    ninf_f = jnp.full((L,), -jnp.inf, jnp.float32)

    def init_body(s, c):
        for j in range(D // L):
            acc[s, pl.ds(j * L, L)] = zeros_f
        for j in range(D // L):
            acc[s, pl.ds(D + j * L, L)] = ninf_f
        return c

    lax.fori_loop(0, L, init_body, 0)
    pltpu.make_async_copy(hist_hbm, table, sem0).wait()

    carry = jnp.int32(0)
    for blk in range(OFFS // L - 1):
        tot = table[pl.ds(blk * L, L)]
        for ww in range(1, NW):
            tot = tot + table[pl.ds(ww * HBINS + blk * L, L)]
        cs = plsc.cumsum(tot)
        offs[pl.ds(blk * L, L)] = cs - tot + carry
        carry = carry + jnp.sum(tot)

    seg0 = w * L
    va = offs[pl.ds(seg0, L)]
    vb = offs[pl.ds(seg0 + L, L)]
    iota = lax.iota(jnp.int32, L)

    def pick(vec, j):
        return jnp.sum(jnp.where(iota == j, vec, 0))

    for j in range(L):
        sm[j] = pick(va, j)
    sm[L] = pick(vb, 0)
    sm[CUR] = 0

    wlo = sm[0]
    whi = sm[L]
    first = (wlo // 8) * 8
    nchunks = (whi - first + CHUNK - 1) // CHUNK

    def issue(k, buf, sem):
        gbase = first + k * CHUNK
        base = jnp.minimum(gbase, N - CHUNK)
        pltpu.async_copy(x_hbm.at[pl.ds(pl.multiple_of(base, 8), CHUNK)], buf, sem)

    @pl.when(nchunks > 0)
    def _prime():
        issue(0, buf0, sem0)

    def process(k, buf):
        gbase = first + k * CHUNK
        base = jnp.minimum(gbase, N - CHUNK)
        cend = jnp.minimum(gbase + CHUNK, whi)

        def cond(cv):
            cur, done = cv
            return jnp.logical_and(
                jnp.logical_not(done),
                jnp.logical_and(cur < L, sm[cur] < cend),
            )

        def body(cv):
            cur, done = cv
            lo_s = jnp.maximum(sm[cur], gbase)
            hi_s = jnp.minimum(sm[cur + 1], cend)
            jlo = lo_s - base
            jhi = hi_s - base
            for h in range(2):
                half = h * (D // 2)

                def row_body(j, rcv):
                    sums, maxs = rcv
                    ns, nm = [], []
                    for cc in range(D // 2 // L):
                        v = buf[j, pl.ds(half + cc * L, L)]
                        ns.append(sums[cc] + v)
                        nm.append(jnp.maximum(maxs[cc], v))
                    return tuple(ns), tuple(nm)

                init = (
                    tuple(zeros_f for _ in range(D // 2 // L)),
                    tuple(ninf_f for _ in range(D // 2 // L)),
                )
                sums, maxs = lax.fori_loop(jlo, jhi, row_body, init)
                for cc in range(D // 2 // L):
                    dss = pl.ds(half + cc * L, L)
                    acc[cur, dss] = acc[cur, dss] + sums[cc]
                    dsm = pl.ds(D + half + cc * L, L)
                    acc[cur, dsm] = jnp.maximum(acc[cur, dsm], maxs[cc])
            adv = sm[cur + 1] <= cend
            return cur + adv.astype(jnp.int32), jnp.logical_not(adv)

        cur0 = sm[CUR]
        curf, _ = lax.while_loop(cond, body, (cur0, jnp.bool_(False)))
        sm[CUR] = curf

    def wait(buf, sem):
        pltpu.make_async_copy(x_hbm.at[pl.ds(0, CHUNK)], buf, sem).wait()

    npairs = (nchunks + 1) // 2

    def pair_body(g, c):
        k0 = 2 * g

        @pl.when(k0 < nchunks)
        def _b0():
            wait(buf0, sem0)

            @pl.when(k0 + 1 < nchunks)
            def _i1():
                issue(k0 + 1, buf1, sem1)

            process(k0, buf0)

        @pl.when(k0 + 1 < nchunks)
        def _b1():
            wait(buf1, sem1)

            @pl.when(k0 + 2 < nchunks)
            def _i2():
                issue(k0 + 2, buf0, sem0)

            process(k0 + 1, buf1)

        return c

    lax.fori_loop(0, npairs, pair_body, 0)

    def fin_body(s, c):
        n = sm[s + 1] - sm[s]
        nf = jnp.broadcast_to(jnp.maximum(n, 1).astype(jnp.float32), (L,))
        scale = jnp.where(n > 0, 1.0, 0.0) / nf
        for j in range(D // L):
            dss = pl.ds(j * L, L)
            acc[s, dss] = acc[s, dss] * scale
        for j in range(D // L):
            dsm = pl.ds(D + j * L, L)
            acc[s, dsm] = jnp.where(n > 0, acc[s, dsm], 0.0)
        return c

    lax.fori_loop(0, L, fin_body, 0)
    pltpu.sync_copy(acc, out_hbm.at[pl.ds(pl.multiple_of(seg0, 8), L)])


def kernel(x, batch):
    batch = batch.astype(jnp.int32)
    hist = _hist_kernel(batch)
    return _main_kernel(x, hist)
